# Initial kernel scaffold; baseline (speedup 1.0000x reference)
import jax
import jax.numpy as jnp
from jax.experimental import pallas as pl


def kernel(inputs):
    raise NotImplementedError("write your pallas kernel here")


# speedup vs baseline: 20.9997x; 20.9997x over previous
import functools

import jax
import jax.numpy as jnp
from jax import lax
from jax.experimental import pallas as pl
from jax.experimental.pallas import tpu as pltpu
from jax.experimental.pallas import tpu_sc as plsc

K_TOP = 8
LANES = 16
NUM_CORES = 2
NUM_SUBCORES = 16
NUM_WORKERS = NUM_CORES * NUM_SUBCORES
CHUNK = 256
UNROLL = 4


def _topk_sc(x3):
    B, S, C = x3.shape
    CG = C // LANES
    GPW = (B * CG) // NUM_WORKERS
    WPB = NUM_WORKERS // B
    NCHUNK = S // CHUNK

    x = x3.reshape(B * S, CG, LANES)
    mesh = plsc.VectorSubcoreMesh(core_axis_name="c", subcore_axis_name="s")

    @functools.partial(
        pl.kernel,
        out_type=jax.ShapeDtypeStruct((B, CG, K_TOP, LANES), jnp.float32),
        mesh=mesh,
        scratch_types=[
            pltpu.VMEM((CHUNK, GPW, LANES), jnp.float32),
            pltpu.VMEM((CHUNK, GPW, LANES), jnp.float32),
            pltpu.VMEM((GPW, K_TOP, LANES), jnp.float32),
            pltpu.SemaphoreType.DMA,
            pltpu.SemaphoreType.DMA,
        ],
        compiler_params=pltpu.CompilerParams(use_tc_tiling_on_sc=False),
    )
    def k(x_hbm, out_hbm, buf0, buf1, acc, sem0, sem1):
        wid = lax.axis_index("s") * NUM_CORES + lax.axis_index("c")
        b = wid // WPB
        g0 = (wid % WPB) * GPW
        row0 = b * S

        neg = jnp.full((LANES,), -jnp.inf, dtype=jnp.float32)
        for g in range(GPW):
            for kk in range(K_TOP):
                acc[g, kk] = neg

        pltpu.async_copy(
            x_hbm.at[pl.ds(row0, CHUNK), pl.ds(g0, GPW)], buf0, sem0)
        pltpu.async_copy(
            x_hbm.at[pl.ds(row0 + CHUNK, CHUNK), pl.ds(g0, GPW)], buf1, sem1)

        def process(buf):
            for g in range(GPW):
                def row_body(i, t, g=g, buf=buf):
                    t = list(t)
                    for u in range(UNROLL):
                        v = buf[i * UNROLL + u, g]
                        for kk in range(K_TOP):
                            hi = jnp.maximum(t[kk], v)
                            v = jnp.minimum(t[kk], v)
                            t[kk] = hi
                    return tuple(t)

                t = tuple(acc[g, kk] for kk in range(K_TOP))
                t = lax.fori_loop(0, CHUNK // UNROLL, row_body, t)
                for kk in range(K_TOP):
                    acc[g, kk] = t[kk]

        @pl.loop(0, NCHUNK, step=2)
        def _(ci):
            for j, (buf, sem) in enumerate(((buf0, sem0), (buf1, sem1))):
                cc = ci + j
                pltpu.make_async_copy(
                    x_hbm.at[pl.ds(row0, CHUNK), pl.ds(g0, GPW)], buf, sem
                ).wait()
                process(buf)

                @pl.when(cc + 2 < NCHUNK)
                def _(buf=buf, sem=sem, cc=cc):
                    pltpu.async_copy(
                        x_hbm.at[pl.ds(row0 + (cc + 2) * CHUNK, CHUNK),
                                 pl.ds(g0, GPW)],
                        buf, sem)

        pltpu.sync_copy(acc, out_hbm.at[b, pl.ds(g0, GPW)])

    return k(x)


def kernel(inputs):
    B, S, C = inputs.shape
    out4 = _topk_sc(inputs)
    return jnp.transpose(out4, (0, 1, 3, 2)).reshape(B, C * K_TOP)


# speedup vs baseline: 29.7989x; 1.4190x over previous
import functools

import jax
import jax.numpy as jnp
from jax import lax
from jax.experimental import pallas as pl
from jax.experimental.pallas import tpu as pltpu
from jax.experimental.pallas import tpu_sc as plsc

K_TOP = 8
LANES = 16
NUM_CORES = 2
NUM_SUBCORES = 16
NUM_WORKERS = NUM_CORES * NUM_SUBCORES
CHUNK = 256
UNROLL = 4
CPW = 128


def _topk_sc(x3):
    B, S, C = x3.shape
    CG = C // LANES
    GPW = CPW // LANES
    WPB = NUM_WORKERS // B
    NCHUNK = S // CHUNK

    x = x3.reshape(B * S, C)
    mesh = plsc.VectorSubcoreMesh(core_axis_name="c", subcore_axis_name="s")

    @functools.partial(
        pl.kernel,
        out_type=jax.ShapeDtypeStruct((B, CG, K_TOP * LANES), jnp.float32),
        mesh=mesh,
        scratch_types=[
            pltpu.VMEM((CHUNK, CPW), jnp.float32),
            pltpu.VMEM((CHUNK, CPW), jnp.float32),
            pltpu.VMEM((GPW, K_TOP * LANES), jnp.float32),
            pltpu.SemaphoreType.DMA,
            pltpu.SemaphoreType.DMA,
        ],
        compiler_params=pltpu.CompilerParams(use_tc_tiling_on_sc=True),
    )
    def k(x_hbm, out_hbm, buf0, buf1, acc, sem0, sem1):
        wid = lax.axis_index("s") * NUM_CORES + lax.axis_index("c")
        b = wid // WPB
        seg = wid % WPB
        g0 = seg * GPW
        c0 = seg * CPW
        row0 = b * S

        neg = jnp.full((LANES,), -jnp.inf, dtype=jnp.float32)
        for g in range(GPW):
            for kk in range(K_TOP):
                acc[g, pl.ds(kk * LANES, LANES)] = neg

        pltpu.async_copy(
            x_hbm.at[pl.ds(row0, CHUNK), pl.ds(c0, CPW)], buf0, sem0)
        pltpu.async_copy(
            x_hbm.at[pl.ds(row0 + CHUNK, CHUNK), pl.ds(c0, CPW)], buf1, sem1)

        def process(buf):
            for g in range(GPW):
                def row_body(i, t, g=g, buf=buf):
                    t = list(t)
                    for u in range(UNROLL):
                        v = buf[i * UNROLL + u, pl.ds(g * LANES, LANES)]
                        for kk in range(K_TOP):
                            hi = jnp.maximum(t[kk], v)
                            v = jnp.minimum(t[kk], v)
                            t[kk] = hi
                    return tuple(t)

                t = tuple(acc[g, pl.ds(kk * LANES, LANES)]
                          for kk in range(K_TOP))
                t = lax.fori_loop(0, CHUNK // UNROLL, row_body, t)
                for kk in range(K_TOP):
                    acc[g, pl.ds(kk * LANES, LANES)] = t[kk]

        @pl.loop(0, NCHUNK, step=2)
        def _(ci):
            for j, (buf, sem) in enumerate(((buf0, sem0), (buf1, sem1))):
                cc = ci + j
                pltpu.make_async_copy(
                    x_hbm.at[pl.ds(row0, CHUNK), pl.ds(c0, CPW)], buf, sem
                ).wait()
                process(buf)

                @pl.when(cc + 2 < NCHUNK)
                def _(buf=buf, sem=sem, cc=cc):
                    pltpu.async_copy(
                        x_hbm.at[pl.ds(row0 + (cc + 2) * CHUNK, CHUNK),
                                 pl.ds(c0, CPW)],
                        buf, sem)

        pltpu.sync_copy(acc, out_hbm.at[b, pl.ds(g0, GPW)])

    return k(x)


def kernel(inputs):
    B, S, C = inputs.shape
    out3 = _topk_sc(inputs)
    out4 = out3.reshape(B, C // LANES, K_TOP, LANES)
    return jnp.transpose(out4, (0, 1, 3, 2)).reshape(B, C * K_TOP)


# speedup vs baseline: 40.9397x; 1.3739x over previous
import functools

import jax
import jax.numpy as jnp
from jax import lax
from jax.experimental import pallas as pl
from jax.experimental.pallas import tpu as pltpu
from jax.experimental.pallas import tpu_sc as plsc

K_TOP = 8
LANES = 16
NUM_CORES = 2
NUM_SUBCORES = 16
NUM_WORKERS = NUM_CORES * NUM_SUBCORES
CHUNK = 256
BLOCK = 16
CPW = 128


def _cas(a, b):
    return jnp.maximum(a, b), jnp.minimum(a, b)


def _merge22(A, B):
    c0, x = _cas(A[0], B[0])
    y, c3 = _cas(A[1], B[1])
    c1, c2 = _cas(x, y)
    return [c0, c1, c2, c3]


def _merge44(A, B):
    E = _merge22([A[0], A[2]], [B[0], B[2]])
    O = _merge22([A[1], A[3]], [B[1], B[3]])
    c1, c2 = _cas(O[0], E[1])
    c3, c4 = _cas(O[1], E[2])
    c5, c6 = _cas(O[2], E[3])
    return [E[0], c1, c2, c3, c4, c5, c6, O[3]]


def _bitonic_clean8(x):
    y = [None] * 8
    for i in range(4):
        y[i], y[i + 4] = _cas(x[i], x[i + 4])
    z = [None] * 8
    for h in (0, 4):
        for i in range(2):
            z[h + i], z[h + i + 2] = _cas(y[h + i], y[h + i + 2])
    w = [None] * 8
    for h in (0, 2, 4, 6):
        w[h], w[h + 1] = _cas(z[h], z[h + 1])
    return w


def _merge_top8(A, B):
    return _bitonic_clean8([jnp.maximum(A[i], B[7 - i]) for i in range(8)])


def _fold_block(T, v):
    S2 = [_cas(v[2 * j], v[2 * j + 1]) for j in range(8)]
    S4 = [_merge22(S2[2 * j], S2[2 * j + 1]) for j in range(4)]
    S8a = _merge44(S4[0], S4[1])
    S8b = _merge44(S4[2], S4[3])
    S = _merge_top8(S8a, S8b)
    return _merge_top8(list(T), S)


def _topk_sc(x3):
    B, S, C = x3.shape
    CG = C // LANES
    GPW = CPW // LANES
    WPB = NUM_WORKERS // B
    NCHUNK = S // CHUNK

    x = x3.reshape(B * S, C)
    mesh = plsc.VectorSubcoreMesh(core_axis_name="c", subcore_axis_name="s")

    @functools.partial(
        pl.kernel,
        out_type=jax.ShapeDtypeStruct((B, CG, K_TOP * LANES), jnp.float32),
        mesh=mesh,
        scratch_types=[
            pltpu.VMEM((CHUNK, CPW), jnp.float32),
            pltpu.VMEM((CHUNK, CPW), jnp.float32),
            pltpu.VMEM((GPW, K_TOP * LANES), jnp.float32),
            pltpu.SemaphoreType.DMA,
            pltpu.SemaphoreType.DMA,
        ],
        compiler_params=pltpu.CompilerParams(use_tc_tiling_on_sc=True),
    )
    def k(x_hbm, out_hbm, buf0, buf1, acc, sem0, sem1):
        wid = lax.axis_index("s") * NUM_CORES + lax.axis_index("c")
        b = wid // WPB
        seg = wid % WPB
        g0 = seg * GPW
        c0 = seg * CPW
        row0 = b * S

        neg = jnp.full((LANES,), -jnp.inf, dtype=jnp.float32)
        for g in range(GPW):
            for kk in range(K_TOP):
                acc[g, pl.ds(kk * LANES, LANES)] = neg

        pltpu.async_copy(
            x_hbm.at[pl.ds(row0, CHUNK), pl.ds(c0, CPW)], buf0, sem0)
        pltpu.async_copy(
            x_hbm.at[pl.ds(row0 + CHUNK, CHUNK), pl.ds(c0, CPW)], buf1, sem1)

        def process(buf):
            for g in range(GPW):
                def blk_body(i, t, g=g, buf=buf):
                    v = [buf[i * BLOCK + u, pl.ds(g * LANES, LANES)]
                         for u in range(BLOCK)]
                    return tuple(_fold_block(t, v))

                t = tuple(acc[g, pl.ds(kk * LANES, LANES)]
                          for kk in range(K_TOP))
                t = lax.fori_loop(0, CHUNK // BLOCK, blk_body, t)
                for kk in range(K_TOP):
                    acc[g, pl.ds(kk * LANES, LANES)] = t[kk]

        @pl.loop(0, NCHUNK, step=2)
        def _(ci):
            for j, (buf, sem) in enumerate(((buf0, sem0), (buf1, sem1))):
                cc = ci + j
                pltpu.make_async_copy(
                    x_hbm.at[pl.ds(row0, CHUNK), pl.ds(c0, CPW)], buf, sem
                ).wait()
                process(buf)

                @pl.when(cc + 2 < NCHUNK)
                def _(buf=buf, sem=sem, cc=cc):
                    pltpu.async_copy(
                        x_hbm.at[pl.ds(row0 + (cc + 2) * CHUNK, CHUNK),
                                 pl.ds(c0, CPW)],
                        buf, sem)

        pltpu.sync_copy(acc, out_hbm.at[b, pl.ds(g0, GPW)])

    return k(x)


def kernel(inputs):
    B, S, C = inputs.shape
    out3 = _topk_sc(inputs)
    out4 = out3.reshape(B, C // LANES, K_TOP, LANES)
    return jnp.transpose(out4, (0, 1, 3, 2)).reshape(B, C * K_TOP)
